# baseline (speedup 1.0000x reference)
import jax
import jax.numpy as jnp
from jax import lax
from jax.experimental import pallas as pl
from jax.experimental.pallas import tpu as pltpu
from jax.experimental.pallas import tpu_sc as plsc

N_NODES = 10000
N_EDGES = 320000
N_GRAPHS = 256
BLKN = 2000
BLKE = 6400
GW = 128
TROWS = 2 * N_NODES + N_GRAPHS

f32 = jnp.float32
bf16 = jnp.bfloat16

_SC_PARAMS = pltpu.CompilerParams(use_tc_tiling_on_sc=False)


def _relu(x):
    return jnp.maximum(x, 0.0)


def _mm(x, w):
    return jnp.dot(x.astype(bf16), w.astype(bf16), preferred_element_type=f32)


def _tca_body(sites_ref, states_ref, sfc1w, sfc1b, sfc2w, sfc2b,
              gfc1w, gfc1b, gfc2w, gfc2b, bu1w, bu1b, su1w, su1b,
              sfeat_ref, T_ref, gfeat_ref, p4g_ref, psu_ref):
    i = pl.program_id(0)
    x = sites_ref[...]
    h = _relu(_mm(x, sfc1w[...]) + sfc1b[...])
    sf = _relu(_mm(h, sfc2w[...]) + sfc2b[...])
    sfeat_ref[...] = sf
    w = bu1w[...]
    T_ref[...] = jnp.concatenate([_mm(sf, w[0:64]), _mm(sf, w[64:128])],
                                 axis=1)

    @pl.when(i == 0)
    def _():
        xs = states_ref[...]
        hg = _relu(_mm(xs, gfc1w[...]) + gfc1b[...])
        gf = _relu(_mm(hg, gfc2w[...]) + gfc2b[...])
        gfeat_ref[...] = gf
        p4g_ref[...] = _mm(gf, w[192:256]) + bu1b[...]
        psu_ref[...] = _mm(gf, su1w[...][128:192]) + su1b[...]


def _tcb_body(bonds_ref, bfc1w, bfc1b, bfc2w, bfc2b, bf_ref):
    x = bonds_ref[...]
    h = _relu(_mm(x, bfc1w[...]) + bfc1b[...])
    bfeat = _relu(_mm(h, bfc2w[...]) + bfc2b[...])
    bf_ref[...] = jnp.concatenate([bfeat[:BLKE // 2], bfeat[BLKE // 2:]],
                                  axis=1)


def _tcc_body(ab_ref, bfp_ref, gp_ref,
              bu1w, bu2w, bu2b, bu3w, bu3b,
              bout_ref, bnew_ref):
    bfp = bfp_ref[...]
    bfeat = jnp.concatenate([bfp[:, 0:64], bfp[:, 64:128]], axis=0)
    gp = gp_ref[...]
    gterm = jnp.concatenate([gp[:, 0:64], gp[:, 64:128]], axis=0)
    w3 = bu1w[...][128:192]
    ab = ab_ref[...]
    h1 = _relu(ab[:, 0:64] + ab[:, 64:128] + _mm(bfeat, w3) + gterm)
    h2 = _relu(_mm(h1, bu2w[...]) + bu2b[...])
    bn = _relu(_mm(h2, bu3w[...]) + bu3b[...])
    bout_ref[...] = (bn + bfeat).T
    bnew_ref[...] = jnp.concatenate([bn[:BLKE // 2], bn[BLKE // 2:]], axis=1)


def _tce_body(nsum_ref, ncnt_ref, sfeat_ref, g2s_ref, psu_ref,
              su1w, su2w, su2b, su3w, su3b,
              sout_ref, spool_ref, scnt_ref):
    i = pl.program_id(0)
    nsum = nsum_ref[0] + nsum_ref[1]
    cnt = ncnt_ref[0, :, 0:1] + ncnt_ref[1, :, 0:1]
    bp = nsum / jnp.maximum(cnt, 1.0)
    sf = sfeat_ref[...]
    g2s = g2s_ref[0, 0, :]
    oh = (lax.broadcasted_iota(jnp.int32, (BLKN, N_GRAPHS), 1)
          == g2s[:, None]).astype(bf16)
    oht = (lax.broadcasted_iota(jnp.int32, (N_GRAPHS, BLKN), 0)
           == g2s[None, :])
    w = su1w[...]
    gterm = jnp.dot(oh, psu_ref[...].astype(bf16), preferred_element_type=f32)
    h = _relu(_mm(bp, w[0:64]) + _mm(sf, w[64:128]) + gterm)
    h = _relu(_mm(h, su2w[...]) + su2b[...])
    sn = _relu(_mm(h, su3w[...]) + su3b[...])
    sout_ref[...] = sn + sf

    @pl.when(i == 0)
    def _():
        spool_ref[...] = jnp.zeros_like(spool_ref)
        scnt_ref[...] = jnp.zeros_like(scnt_ref)

    spool_ref[...] += jnp.dot(oht.astype(bf16), sn.astype(bf16),
                              preferred_element_type=f32)
    scnt_ref[...] += jnp.sum(oht.astype(f32), axis=1, keepdims=True)


def _tcf_body(gsum_ref, gcnt_ref, spool_ref, scnt_ref, gfeat_ref,
              xu1w, xu1b, xu2w, xu2b, xu3w, xu3b, gout_ref):
    bsum = gsum_ref[0] + gsum_ref[1]
    bc = gcnt_ref[0, :, 0:1] + gcnt_ref[1, :, 0:1]
    bp = bsum / jnp.maximum(bc, 1.0)
    sp = spool_ref[...] / jnp.maximum(scnt_ref[...], 1.0)
    gf = gfeat_ref[...]
    w = xu1w[...]
    h = _relu(_mm(bp, w[0:64]) + _mm(sp, w[64:128]) + _mm(gf, w[128:192])
              + xu1b[...])
    h = _relu(_mm(h, xu2w[...]) + xu2b[...])
    gn = _relu(_mm(h, xu3w[...]) + xu3b[...])
    gout_ref[...] = gn + gf


def _sc_gather(table, idx_ab, idx_gp, idx_n, idx_g):
    mesh = plsc.VectorSubcoreMesh(core_axis_name="c", subcore_axis_name="s")
    srows = TROWS // 16
    NROWS = N_NODES // 16
    GROWS = N_GRAPHS // 16

    def body(table_hbm, iab_hbm, igp_hbm, in_hbm, ig_hbm,
             out_ab, out_gp, ncnt_hbm, gcnt_hbm,
             table_sh, ncnt_sh, gcnt_sh, ones_v, zrow16_v):
        cid = lax.axis_index("c")
        sid = lax.axis_index("s")

        pltpu.sync_copy(table_hbm.at[pl.ds(sid * srows, srows)],
                        table_sh.at[pl.ds(sid * srows, srows)])

        @pl.loop(0, GW)
        def _(r):
            ones_v[pl.ds(r, 1), pl.ds(0, 16)] = jnp.ones((1, 16), f32)

        @pl.loop(0, 125)
        def _(r):
            zrow16_v[pl.ds(r, 1), pl.ds(0, 16)] = jnp.zeros((1, 16), f32)

        @pl.loop(0, 5)
        def _(k):
            pltpu.sync_copy(zrow16_v, ncnt_sh.at[pl.ds(sid * NROWS + k * 125,
                                                       125)])

        pltpu.sync_copy(zrow16_v.at[pl.ds(0, GROWS)],
                        gcnt_sh.at[pl.ds(sid * GROWS, GROWS)])

        plsc.subcore_barrier()

        def g_ab(i_vmem, o_vmem):
            pltpu.sync_copy(table_sh.at[i_vmem.at[0]], o_vmem)

        pltpu.emit_pipeline(
            g_ab,
            grid=(2 * N_EDGES // GW,),
            in_specs=[pl.BlockSpec((1, GW), lambda i: (0, i))],
            out_specs=[pl.BlockSpec((GW, 64), lambda i: (i, 0))],
            core_axis_name=("c", "s"),
            dimension_semantics=(pltpu.PARALLEL,),
        )(iab_hbm, out_ab)

        def g_gp(i_vmem, o_vmem):
            pltpu.sync_copy(table_sh.at[i_vmem.at[0]], o_vmem)

        pltpu.emit_pipeline(
            g_gp,
            grid=(N_EDGES // GW,),
            in_specs=[pl.BlockSpec((1, GW), lambda i: (0, i))],
            out_specs=[pl.BlockSpec((GW, 64), lambda i: (i, 0))],
            core_axis_name=("c", "s"),
            dimension_semantics=(pltpu.PARALLEL,),
        )(igp_hbm, out_gp)

        def cnts(in_vmem, ig_vmem):
            pltpu.sync_copy(ones_v, ncnt_sh.at[in_vmem.at[0]], add=True)
            pltpu.sync_copy(ones_v, gcnt_sh.at[ig_vmem.at[0]], add=True)

        pltpu.emit_pipeline(
            cnts,
            grid=(N_EDGES // GW,),
            in_specs=[pl.BlockSpec((1, GW), lambda i: (0, i)),
                      pl.BlockSpec((1, GW), lambda i: (0, i))],
            out_specs=[],
            core_axis_name=("c", "s"),
            dimension_semantics=(pltpu.PARALLEL,),
        )(in_hbm, ig_hbm)

        plsc.subcore_barrier()

        pltpu.sync_copy(ncnt_sh.at[pl.ds(sid * NROWS, NROWS)],
                        ncnt_hbm.at[cid].at[pl.ds(sid * NROWS, NROWS)])
        pltpu.sync_copy(gcnt_sh.at[pl.ds(sid * GROWS, GROWS)],
                        gcnt_hbm.at[cid].at[pl.ds(sid * GROWS, GROWS)])

    fn = pl.kernel(
        body,
        out_type=[jax.ShapeDtypeStruct((2 * N_EDGES, 64), f32),
                  jax.ShapeDtypeStruct((N_EDGES, 64), f32),
                  jax.ShapeDtypeStruct((2, N_NODES, 16), f32),
                  jax.ShapeDtypeStruct((2, N_GRAPHS, 16), f32)],
        mesh=mesh, compiler_params=_SC_PARAMS,
        scratch_types=[
            pltpu.VMEM_SHARED((TROWS, 64), f32),
            pltpu.VMEM_SHARED((N_NODES, 16), f32),
            pltpu.VMEM_SHARED((N_GRAPHS, 16), f32),
            pltpu.VMEM((GW, 16), f32),
            pltpu.VMEM((125, 16), f32),
        ],
    )
    return fn(table, idx_ab, idx_gp, idx_n, idx_g)


def _sc_scatter(bnew, idx_n, idx_g):
    mesh = plsc.VectorSubcoreMesh(core_axis_name="c", subcore_axis_name="s")
    NROWS = N_NODES // 16
    GROWS = N_GRAPHS // 16

    def body(bnew_hbm, in_hbm, ig_hbm, nsum_hbm, gsum_hbm,
             acc_sh, gacc_sh, zrow_v):
        cid = lax.axis_index("c")
        sid = lax.axis_index("s")

        @pl.loop(0, 125)
        def _(r):
            @pl.loop(0, 64, step=16)
            def _(c2):
                zrow_v[pl.ds(r, 1), pl.ds(c2, 16)] = jnp.zeros((1, 16), f32)

        @pl.loop(0, 5)
        def _(k):
            pltpu.sync_copy(zrow_v, acc_sh.at[pl.ds(sid * NROWS + k * 125,
                                                    125)])

        pltpu.sync_copy(zrow_v.at[pl.ds(0, GROWS)],
                        gacc_sh.at[pl.ds(sid * GROWS, GROWS)])

        plsc.subcore_barrier()

        def inner(v_vmem, in_vmem, ig_vmem):
            pltpu.sync_copy(v_vmem, acc_sh.at[in_vmem.at[0]], add=True)
            pltpu.sync_copy(v_vmem, gacc_sh.at[ig_vmem.at[0]], add=True)

        pltpu.emit_pipeline(
            inner,
            grid=(N_EDGES // GW,),
            in_specs=[pl.BlockSpec((GW, 64), lambda i: (i, 0)),
                      pl.BlockSpec((1, GW), lambda i: (0, i)),
                      pl.BlockSpec((1, GW), lambda i: (0, i))],
            out_specs=[],
            core_axis_name=("c", "s"),
            dimension_semantics=(pltpu.PARALLEL,),
        )(bnew_hbm, in_hbm, ig_hbm)

        plsc.subcore_barrier()

        pltpu.sync_copy(acc_sh.at[pl.ds(sid * NROWS, NROWS)],
                        nsum_hbm.at[cid].at[pl.ds(sid * NROWS, NROWS)])
        pltpu.sync_copy(gacc_sh.at[pl.ds(sid * GROWS, GROWS)],
                        gsum_hbm.at[cid].at[pl.ds(sid * GROWS, GROWS)])

    fn = pl.kernel(
        body,
        out_type=[jax.ShapeDtypeStruct((2, N_NODES, 64), f32),
                  jax.ShapeDtypeStruct((2, N_GRAPHS, 64), f32)],
        mesh=mesh, compiler_params=_SC_PARAMS,
        scratch_types=[
            pltpu.VMEM_SHARED((N_NODES, 64), f32),
            pltpu.VMEM_SHARED((N_GRAPHS, 64), f32),
            pltpu.VMEM((125, 64), f32),
        ],
    )
    return fn(bnew, idx_n, idx_g)


def kernel(sites, bonds, states, indices1, indices2, graph_to_sites,
           graph_to_bonds, bfc1_W, bfc1_b, bfc2_W, bfc2_b, sfc1_W, sfc1_b,
           sfc2_W, sfc2_b, gfc1_W, gfc1_b, gfc2_W, gfc2_b, bu1_W, bu1_b,
           bu2_W, bu2_b, bu3_W, bu3_b, su1_W, su1_b, su2_W, su2_b, su3_W,
           su3_b, xu1_W, xu1_b, xu2_W, xu2_b, xu3_W, xu3_b):
    r2 = lambda b: b.reshape(1, -1)
    i32 = jnp.int32
    idx1 = indices1.astype(i32)
    idx2 = indices2.astype(i32)
    g2b = graph_to_bonds.astype(i32)
    g2s = graph_to_sites.astype(i32)

    n_nblk = N_NODES // BLKN
    n_eblk = N_EDGES // BLKE
    const = lambda shp: pl.BlockSpec(shp, lambda i: tuple(0 for _ in shp))
    perm = lambda a: a.reshape(n_eblk, 2, BLKE // 2).transpose(0, 2, 1)

    sfeat, T, gfeat, p4g, psu = pl.pallas_call(
        _tca_body,
        grid=(n_nblk,),
        in_specs=[
            pl.BlockSpec((BLKN, 128), lambda i: (i, 0)),
            const((N_GRAPHS, 128)),
            const((128, 64)), const((1, 64)), const((64, 64)), const((1, 64)),
            const((128, 64)), const((1, 64)), const((64, 64)), const((1, 64)),
            const((256, 64)), const((1, 64)), const((192, 64)), const((1, 64)),
        ],
        out_specs=[
            pl.BlockSpec((BLKN, 64), lambda i: (i, 0)),
            pl.BlockSpec((BLKN, 128), lambda i: (i, 0)),
            const((N_GRAPHS, 64)),
            const((N_GRAPHS, 64)),
            const((N_GRAPHS, 64)),
        ],
        out_shape=[
            jax.ShapeDtypeStruct((N_NODES, 64), f32),
            jax.ShapeDtypeStruct((N_NODES, 128), f32),
            jax.ShapeDtypeStruct((N_GRAPHS, 64), f32),
            jax.ShapeDtypeStruct((N_GRAPHS, 64), f32),
            jax.ShapeDtypeStruct((N_GRAPHS, 64), f32),
        ],
    )(sites, states, sfc1_W, r2(sfc1_b), sfc2_W, r2(sfc2_b),
      gfc1_W, r2(gfc1_b), gfc2_W, r2(gfc2_b), bu1_W, r2(bu1_b),
      su1_W, r2(su1_b))

    table = jnp.concatenate([T.reshape(2 * N_NODES, 64), p4g], axis=0)
    idx_ab = jnp.stack([2 * idx1, 2 * idx2 + 1], axis=1).reshape(1, -1)
    idx_gp = (2 * N_NODES + perm(g2b)).reshape(1, -1)
    ab, gp, ncnt, gcnt = _sc_gather(table, idx_ab, idx_gp,
                                    idx1.reshape(1, -1), g2b.reshape(1, -1))
    ab = ab.reshape(N_EDGES, 128)
    gp = gp.reshape(N_EDGES // 2, 128)

    bfp = pl.pallas_call(
        _tcb_body,
        grid=(n_eblk,),
        in_specs=[
            pl.BlockSpec((BLKE, 128), lambda i: (i, 0)),
            const((128, 64)), const((1, 64)), const((64, 64)), const((1, 64)),
        ],
        out_specs=pl.BlockSpec((BLKE // 2, 128), lambda i: (i, 0)),
        out_shape=jax.ShapeDtypeStruct((N_EDGES // 2, 128), f32),
    )(bonds, bfc1_W, r2(bfc1_b), bfc2_W, r2(bfc2_b))

    bout, bnew = pl.pallas_call(
        _tcc_body,
        grid=(n_eblk,),
        in_specs=[
            pl.BlockSpec((BLKE, 128), lambda i: (i, 0)),
            pl.BlockSpec((BLKE // 2, 128), lambda i: (i, 0)),
            pl.BlockSpec((BLKE // 2, 128), lambda i: (i, 0)),
            const((256, 64)), const((64, 64)), const((1, 64)),
            const((64, 64)), const((1, 64)),
        ],
        out_specs=[
            pl.BlockSpec((64, BLKE), lambda i: (0, i)),
            pl.BlockSpec((BLKE // 2, 128), lambda i: (i, 0)),
        ],
        out_shape=[
            jax.ShapeDtypeStruct((64, N_EDGES), f32),
            jax.ShapeDtypeStruct((N_EDGES // 2, 128), f32),
        ],
    )(ab, bfp, gp, bu1_W, bu2_W, r2(bu2_b), bu3_W, r2(bu3_b))
    bout = bout.T

    idx_sc = perm(idx1).reshape(1, -1)
    idx_gc = perm(g2b).reshape(1, -1)
    nsum, gsum = _sc_scatter(bnew.reshape(N_EDGES, 64), idx_sc, idx_gc)

    g2s3 = g2s.reshape(n_nblk, 1, BLKN)
    sout, spool_s, scnt = pl.pallas_call(
        _tce_body,
        grid=(n_nblk,),
        in_specs=[
            pl.BlockSpec((2, BLKN, 64), lambda i: (0, i, 0)),
            pl.BlockSpec((2, BLKN, 16), lambda i: (0, i, 0)),
            pl.BlockSpec((BLKN, 64), lambda i: (i, 0)),
            pl.BlockSpec((1, 1, BLKN), lambda i: (i, 0, 0)),
            const((N_GRAPHS, 64)),
            const((192, 64)), const((64, 64)), const((1, 64)),
            const((64, 64)), const((1, 64)),
        ],
        out_specs=[
            pl.BlockSpec((BLKN, 64), lambda i: (i, 0)),
            const((N_GRAPHS, 64)),
            const((N_GRAPHS, 1)),
        ],
        out_shape=[
            jax.ShapeDtypeStruct((N_NODES, 64), f32),
            jax.ShapeDtypeStruct((N_GRAPHS, 64), f32),
            jax.ShapeDtypeStruct((N_GRAPHS, 1), f32),
        ],
    )(nsum, ncnt, sfeat, g2s3, psu, su1_W, su2_W, r2(su2_b),
      su3_W, r2(su3_b))

    gout = pl.pallas_call(
        _tcf_body,
        grid=(1,),
        in_specs=[
            const((2, N_GRAPHS, 64)), const((2, N_GRAPHS, 16)),
            const((N_GRAPHS, 64)), const((N_GRAPHS, 1)),
            const((N_GRAPHS, 64)),
            const((192, 64)), const((1, 64)), const((64, 64)), const((1, 64)),
            const((64, 64)), const((1, 64)),
        ],
        out_specs=const((N_GRAPHS, 64)),
        out_shape=jax.ShapeDtypeStruct((N_GRAPHS, 64), f32),
    )(gsum, gcnt, spool_s, scnt, gfeat, xu1_W, r2(xu1_b),
      xu2_W, r2(xu2_b), xu3_W, r2(xu3_b))

    return sout, bout, gout


# speedup vs baseline: 1.0155x; 1.0155x over previous
import jax
import jax.numpy as jnp
from jax import lax
from jax.experimental import pallas as pl
from jax.experimental.pallas import tpu as pltpu
from jax.experimental.pallas import tpu_sc as plsc

N_NODES = 10000
N_EDGES = 320000
N_GRAPHS = 256
BLKN = 2000
BLKE = 6400
GW = 128

f32 = jnp.float32
bf16 = jnp.bfloat16

_SC_PARAMS = pltpu.CompilerParams(use_tc_tiling_on_sc=False)


def _relu(x):
    return jnp.maximum(x, 0.0)


def _mm(x, w):
    return jnp.dot(x.astype(bf16), w.astype(bf16), preferred_element_type=f32)


def _tca_body(sites_ref, states_ref, sfc1w, sfc1b, sfc2w, sfc2b,
              gfc1w, gfc1b, gfc2w, gfc2b, bu1w, bu1b, su1w, su1b,
              sfeat_ref, T_ref, gfeat_ref, p4g_ref, psu_ref):
    i = pl.program_id(0)
    x = sites_ref[...]
    h = _relu(_mm(x, sfc1w[...]) + sfc1b[...])
    sf = _relu(_mm(h, sfc2w[...]) + sfc2b[...])
    sfeat_ref[...] = sf
    w = bu1w[...]
    T_ref[...] = jnp.concatenate([_mm(sf, w[0:64]), _mm(sf, w[64:128])],
                                 axis=1)

    @pl.when(i == 0)
    def _():
        xs = states_ref[...]
        hg = _relu(_mm(xs, gfc1w[...]) + gfc1b[...])
        gf = _relu(_mm(hg, gfc2w[...]) + gfc2b[...])
        gfeat_ref[...] = gf
        p4g_ref[...] = _mm(gf, w[192:256]) + bu1b[...]
        psu_ref[...] = _mm(gf, su1w[...][128:192]) + su1b[...]


def _tcb_body(bonds_ref, bfc1w, bfc1b, bfc2w, bfc2b, bf_ref):
    x = bonds_ref[...]
    h = _relu(_mm(x, bfc1w[...]) + bfc1b[...])
    bfeat = _relu(_mm(h, bfc2w[...]) + bfc2b[...])
    bf_ref[...] = jnp.concatenate([bfeat[:BLKE // 2], bfeat[BLKE // 2:]],
                                  axis=1)


def _tcc_body(ab_ref, bfp_ref, g2b_ref, p4g_ref,
              bu1w, bu2w, bu2b, bu3w, bu3b,
              bout_ref, bnew_ref, bcnt_ref):
    i = pl.program_id(0)
    bfp = bfp_ref[...]
    bfeat = jnp.concatenate([bfp[:, 0:64], bfp[:, 64:128]], axis=0)
    g2b = g2b_ref[0, 0, :]
    ohb = (lax.broadcasted_iota(jnp.int32, (BLKE, N_GRAPHS), 1)
           == g2b[:, None])
    gterm = jnp.dot(ohb.astype(bf16), p4g_ref[...].astype(bf16),
                    preferred_element_type=f32)
    w3 = bu1w[...][128:192]
    ab = ab_ref[...]
    h1 = _relu(ab[:, 0:64] + ab[:, 64:128] + _mm(bfeat, w3) + gterm)
    h2 = _relu(_mm(h1, bu2w[...]) + bu2b[...])
    bn = _relu(_mm(h2, bu3w[...]) + bu3b[...])
    bout_ref[...] = (bn + bfeat).T
    bnew_ref[...] = jnp.concatenate([bn[:BLKE // 2], bn[BLKE // 2:]], axis=1)

    @pl.when(i == 0)
    def _():
        bcnt_ref[...] = jnp.zeros_like(bcnt_ref)

    bcnt_ref[...] += jnp.sum(ohb.astype(f32), axis=0, keepdims=True)


def _tce_body(nsum_ref, ncnt_ref, sfeat_ref, g2s_ref, psu_ref,
              su1w, su2w, su2b, su3w, su3b,
              sout_ref, spool_ref, scnt_ref):
    i = pl.program_id(0)
    nsum = nsum_ref[0] + nsum_ref[1]
    cnt = ncnt_ref[0, :, 0:1] + ncnt_ref[1, :, 0:1]
    bp = nsum / jnp.maximum(cnt, 1.0)
    sf = sfeat_ref[...]
    g2s = g2s_ref[0, 0, :]
    oh = (lax.broadcasted_iota(jnp.int32, (BLKN, N_GRAPHS), 1)
          == g2s[:, None]).astype(bf16)
    oht = (lax.broadcasted_iota(jnp.int32, (N_GRAPHS, BLKN), 0)
           == g2s[None, :])
    w = su1w[...]
    gterm = jnp.dot(oh, psu_ref[...].astype(bf16), preferred_element_type=f32)
    h = _relu(_mm(bp, w[0:64]) + _mm(sf, w[64:128]) + gterm)
    h = _relu(_mm(h, su2w[...]) + su2b[...])
    sn = _relu(_mm(h, su3w[...]) + su3b[...])
    sout_ref[...] = sn + sf

    @pl.when(i == 0)
    def _():
        spool_ref[...] = jnp.zeros_like(spool_ref)
        scnt_ref[...] = jnp.zeros_like(scnt_ref)

    spool_ref[...] += jnp.dot(oht.astype(bf16), sn.astype(bf16),
                              preferred_element_type=f32)
    scnt_ref[...] += jnp.sum(oht.astype(f32), axis=1, keepdims=True)


def _tcf_body(gsum_ref, bcnt_ref, spool_ref, scnt_ref, gfeat_ref,
              xu1w, xu1b, xu2w, xu2b, xu3w, xu3b, gout_ref):
    bp = (gsum_ref[0] + gsum_ref[1]) / jnp.maximum(bcnt_ref[...], 1.0)
    sp = spool_ref[...] / jnp.maximum(scnt_ref[...], 1.0)
    gf = gfeat_ref[...]
    w = xu1w[...]
    h = _relu(_mm(bp, w[0:64]) + _mm(sp, w[64:128]) + _mm(gf, w[128:192])
              + xu1b[...])
    h = _relu(_mm(h, xu2w[...]) + xu2b[...])
    gn = _relu(_mm(h, xu3w[...]) + xu3b[...])
    gout_ref[...] = gn + gf


def _sc_gather(table, idx):
    n_idx = idx.shape[1]
    n_rows = table.shape[0]
    srows = n_rows // 16
    mesh = plsc.VectorSubcoreMesh(core_axis_name="c", subcore_axis_name="s")

    def body(table_hbm, idx_hbm, out_hbm, table_sh):
        sid = lax.axis_index("s")
        pltpu.sync_copy(table_hbm.at[pl.ds(sid * srows, srows)],
                        table_sh.at[pl.ds(sid * srows, srows)])
        plsc.subcore_barrier()

        def inner(i_vmem, o_vmem):
            pltpu.sync_copy(table_sh.at[i_vmem.at[0]], o_vmem)

        pltpu.emit_pipeline(
            inner,
            grid=(n_idx // GW,),
            in_specs=[pl.BlockSpec((1, GW), lambda i: (0, i))],
            out_specs=[pl.BlockSpec((GW, 64), lambda i: (i, 0))],
            core_axis_name=("c", "s"),
            dimension_semantics=(pltpu.PARALLEL,),
        )(idx_hbm, out_hbm)

    fn = pl.kernel(body, out_type=jax.ShapeDtypeStruct((n_idx, 64), f32),
                   mesh=mesh, compiler_params=_SC_PARAMS,
                   scratch_types=[pltpu.VMEM_SHARED((n_rows, 64), f32)])
    return fn(table, idx)


def _sc_scatter(bnew, idx, idx_g):
    mesh = plsc.VectorSubcoreMesh(core_axis_name="c", subcore_axis_name="s")
    NSUB = 16
    ROWS = N_NODES // NSUB
    GROWS = N_GRAPHS // NSUB

    def body(bnew_hbm, idx_hbm, idxg_hbm, nsum_hbm, ncnt_hbm, gsum_hbm,
             acc_sh, cnt_sh, gacc_sh, ones_v, zrow_v, zrow16_v):
        cid = lax.axis_index("c")
        sid = lax.axis_index("s")

        @pl.loop(0, GW)
        def _(r):
            ones_v[pl.ds(r, 1), pl.ds(0, 16)] = jnp.ones((1, 16), f32)

        @pl.loop(0, 125)
        def _(r):
            @pl.loop(0, 64, step=16)
            def _(c2):
                zrow_v[pl.ds(r, 1), pl.ds(c2, 16)] = jnp.zeros((1, 16), f32)

            zrow16_v[pl.ds(r, 1), pl.ds(0, 16)] = jnp.zeros((1, 16), f32)

        @pl.loop(0, 5)
        def _(k):
            base = sid * ROWS + k * 125
            pltpu.sync_copy(zrow_v, acc_sh.at[pl.ds(base, 125)])
            pltpu.sync_copy(zrow16_v, cnt_sh.at[pl.ds(base, 125)])

        pltpu.sync_copy(zrow_v.at[pl.ds(0, GROWS)],
                        gacc_sh.at[pl.ds(sid * GROWS, GROWS)])

        plsc.subcore_barrier()

        def inner(v_vmem, i_vmem, ig_vmem):
            pltpu.sync_copy(v_vmem, acc_sh.at[i_vmem.at[0]], add=True)
            pltpu.sync_copy(ones_v, cnt_sh.at[i_vmem.at[0]], add=True)
            pltpu.sync_copy(v_vmem, gacc_sh.at[ig_vmem.at[0]], add=True)

        pltpu.emit_pipeline(
            inner,
            grid=(N_EDGES // GW,),
            in_specs=[pl.BlockSpec((GW, 64), lambda i: (i, 0)),
                      pl.BlockSpec((1, GW), lambda i: (0, i)),
                      pl.BlockSpec((1, GW), lambda i: (0, i))],
            out_specs=[],
            core_axis_name=("c", "s"),
            dimension_semantics=(pltpu.PARALLEL,),
        )(bnew_hbm, idx_hbm, idxg_hbm)

        plsc.subcore_barrier()

        pltpu.sync_copy(acc_sh.at[pl.ds(sid * ROWS, ROWS)],
                        nsum_hbm.at[cid].at[pl.ds(sid * ROWS, ROWS)])
        pltpu.sync_copy(cnt_sh.at[pl.ds(sid * ROWS, ROWS)],
                        ncnt_hbm.at[cid].at[pl.ds(sid * ROWS, ROWS)])
        pltpu.sync_copy(gacc_sh.at[pl.ds(sid * GROWS, GROWS)],
                        gsum_hbm.at[cid].at[pl.ds(sid * GROWS, GROWS)])

    fn = pl.kernel(
        body,
        out_type=[jax.ShapeDtypeStruct((2, N_NODES, 64), f32),
                  jax.ShapeDtypeStruct((2, N_NODES, 16), f32),
                  jax.ShapeDtypeStruct((2, N_GRAPHS, 64), f32)],
        mesh=mesh,
        compiler_params=_SC_PARAMS,
        scratch_types=[
            pltpu.VMEM_SHARED((N_NODES, 64), f32),
            pltpu.VMEM_SHARED((N_NODES, 16), f32),
            pltpu.VMEM_SHARED((N_GRAPHS, 64), f32),
            pltpu.VMEM((GW, 16), f32),
            pltpu.VMEM((125, 64), f32),
            pltpu.VMEM((125, 16), f32),
        ],
    )
    return fn(bnew, idx, idx_g)


def kernel(sites, bonds, states, indices1, indices2, graph_to_sites,
           graph_to_bonds, bfc1_W, bfc1_b, bfc2_W, bfc2_b, sfc1_W, sfc1_b,
           sfc2_W, sfc2_b, gfc1_W, gfc1_b, gfc2_W, gfc2_b, bu1_W, bu1_b,
           bu2_W, bu2_b, bu3_W, bu3_b, su1_W, su1_b, su2_W, su2_b, su3_W,
           su3_b, xu1_W, xu1_b, xu2_W, xu2_b, xu3_W, xu3_b):
    r2 = lambda b: b.reshape(1, -1)
    i32 = jnp.int32
    idx1 = indices1.astype(i32)
    idx2 = indices2.astype(i32)
    g2b = graph_to_bonds.astype(i32)
    g2s = graph_to_sites.astype(i32)

    n_nblk = N_NODES // BLKN
    n_eblk = N_EDGES // BLKE
    const = lambda shp: pl.BlockSpec(shp, lambda i: tuple(0 for _ in shp))

    sfeat, T, gfeat, p4g, psu = pl.pallas_call(
        _tca_body,
        grid=(n_nblk,),
        in_specs=[
            pl.BlockSpec((BLKN, 128), lambda i: (i, 0)),
            const((N_GRAPHS, 128)),
            const((128, 64)), const((1, 64)), const((64, 64)), const((1, 64)),
            const((128, 64)), const((1, 64)), const((64, 64)), const((1, 64)),
            const((256, 64)), const((1, 64)), const((192, 64)), const((1, 64)),
        ],
        out_specs=[
            pl.BlockSpec((BLKN, 64), lambda i: (i, 0)),
            pl.BlockSpec((BLKN, 128), lambda i: (i, 0)),
            const((N_GRAPHS, 64)),
            const((N_GRAPHS, 64)),
            const((N_GRAPHS, 64)),
        ],
        out_shape=[
            jax.ShapeDtypeStruct((N_NODES, 64), f32),
            jax.ShapeDtypeStruct((N_NODES, 128), f32),
            jax.ShapeDtypeStruct((N_GRAPHS, 64), f32),
            jax.ShapeDtypeStruct((N_GRAPHS, 64), f32),
            jax.ShapeDtypeStruct((N_GRAPHS, 64), f32),
        ],
    )(sites, states, sfc1_W, r2(sfc1_b), sfc2_W, r2(sfc2_b),
      gfc1_W, r2(gfc1_b), gfc2_W, r2(gfc2_b), bu1_W, r2(bu1_b),
      su1_W, r2(su1_b))

    table = T.reshape(2 * N_NODES, 64)
    idx_all = jnp.stack([2 * idx1, 2 * idx2 + 1], axis=1).reshape(1, -1)
    ab = _sc_gather(table, idx_all).reshape(N_EDGES, 128)

    bfp = pl.pallas_call(
        _tcb_body,
        grid=(n_eblk,),
        in_specs=[
            pl.BlockSpec((BLKE, 128), lambda i: (i, 0)),
            const((128, 64)), const((1, 64)), const((64, 64)), const((1, 64)),
        ],
        out_specs=pl.BlockSpec((BLKE // 2, 128), lambda i: (i, 0)),
        out_shape=jax.ShapeDtypeStruct((N_EDGES // 2, 128), f32),
    )(bonds, bfc1_W, r2(bfc1_b), bfc2_W, r2(bfc2_b))

    g2b3 = g2b.reshape(n_eblk, 1, BLKE)
    bout, bnew, bcnt = pl.pallas_call(
        _tcc_body,
        grid=(n_eblk,),
        in_specs=[
            pl.BlockSpec((BLKE, 128), lambda i: (i, 0)),
            pl.BlockSpec((BLKE // 2, 128), lambda i: (i, 0)),
            pl.BlockSpec((1, 1, BLKE), lambda i: (i, 0, 0)),
            const((N_GRAPHS, 64)),
            const((256, 64)), const((64, 64)), const((1, 64)),
            const((64, 64)), const((1, 64)),
        ],
        out_specs=[
            pl.BlockSpec((64, BLKE), lambda i: (0, i)),
            pl.BlockSpec((BLKE // 2, 128), lambda i: (i, 0)),
            const((1, N_GRAPHS)),
        ],
        out_shape=[
            jax.ShapeDtypeStruct((64, N_EDGES), f32),
            jax.ShapeDtypeStruct((N_EDGES // 2, 128), f32),
            jax.ShapeDtypeStruct((1, N_GRAPHS), f32),
        ],
    )(ab, bfp, g2b3, p4g, bu1_W, bu2_W, r2(bu2_b), bu3_W, r2(bu3_b))
    bout = bout.T

    perm = lambda a: (a.reshape(n_eblk, 2, BLKE // 2)
                      .transpose(0, 2, 1).reshape(1, -1))
    nsum, ncnt, gsum = _sc_scatter(bnew.reshape(N_EDGES, 64), perm(idx1),
                                   perm(g2b))

    g2s3 = g2s.reshape(n_nblk, 1, BLKN)
    sout, spool_s, scnt = pl.pallas_call(
        _tce_body,
        grid=(n_nblk,),
        in_specs=[
            pl.BlockSpec((2, BLKN, 64), lambda i: (0, i, 0)),
            pl.BlockSpec((2, BLKN, 16), lambda i: (0, i, 0)),
            pl.BlockSpec((BLKN, 64), lambda i: (i, 0)),
            pl.BlockSpec((1, 1, BLKN), lambda i: (i, 0, 0)),
            const((N_GRAPHS, 64)),
            const((192, 64)), const((64, 64)), const((1, 64)),
            const((64, 64)), const((1, 64)),
        ],
        out_specs=[
            pl.BlockSpec((BLKN, 64), lambda i: (i, 0)),
            const((N_GRAPHS, 64)),
            const((N_GRAPHS, 1)),
        ],
        out_shape=[
            jax.ShapeDtypeStruct((N_NODES, 64), f32),
            jax.ShapeDtypeStruct((N_GRAPHS, 64), f32),
            jax.ShapeDtypeStruct((N_GRAPHS, 1), f32),
        ],
    )(nsum, ncnt, sfeat, g2s3, psu, su1_W, su2_W, r2(su2_b),
      su3_W, r2(su3_b))

    gout = pl.pallas_call(
        _tcf_body,
        grid=(1,),
        in_specs=[
            const((2, N_GRAPHS, 64)), const((N_GRAPHS, 1)),
            const((N_GRAPHS, 64)), const((N_GRAPHS, 1)),
            const((N_GRAPHS, 64)),
            const((192, 64)), const((1, 64)), const((64, 64)), const((1, 64)),
            const((64, 64)), const((1, 64)),
        ],
        out_specs=const((N_GRAPHS, 64)),
        out_shape=jax.ShapeDtypeStruct((N_GRAPHS, 64), f32),
    )(gsum, bcnt.T, spool_s, scnt, gfeat, xu1_W, r2(xu1_b),
      xu2_W, r2(xu2_b), xu3_W, r2(xu3_b))

    return sout, bout, gout


# speedup vs baseline: 1.1676x; 1.1497x over previous
import jax
import jax.numpy as jnp
from jax import lax
from jax.experimental import pallas as pl
from jax.experimental.pallas import tpu as pltpu
from jax.experimental.pallas import tpu_sc as plsc

N_NODES = 10000
N_EDGES = 320000
N_GRAPHS = 256
BLKN = 2000
BLKE = 6400
GW = 128

f32 = jnp.float32
bf16 = jnp.bfloat16

_SC_PARAMS = pltpu.CompilerParams(use_tc_tiling_on_sc=False)


def _relu(x):
    return jnp.maximum(x, 0.0)


def _mm(x, w):
    return jnp.dot(x.astype(bf16), w.astype(bf16), preferred_element_type=f32)


def _tca_body(sites_ref, states_ref, sfc1w, sfc1b, sfc2w, sfc2b,
              gfc1w, gfc1b, gfc2w, gfc2b, bu1w, bu1b, su1w, su1b,
              sfeat_ref, T_ref, gfeat_ref, p4g_ref, psu_ref):
    i = pl.program_id(0)
    x = sites_ref[...]
    h = _relu(_mm(x, sfc1w[...]) + sfc1b[...])
    sf = _relu(_mm(h, sfc2w[...]) + sfc2b[...])
    sfeat_ref[...] = sf
    w = bu1w[...]
    T_ref[...] = jnp.concatenate([_mm(sf, w[0:64]), _mm(sf, w[64:128])],
                                 axis=1)

    @pl.when(i == 0)
    def _():
        xs = states_ref[...]
        hg = _relu(_mm(xs, gfc1w[...]) + gfc1b[...])
        gf = _relu(_mm(hg, gfc2w[...]) + gfc2b[...])
        gfeat_ref[...] = gf
        p4g_ref[...] = _mm(gf, w[192:256]) + bu1b[...]
        psu_ref[...] = _mm(gf, su1w[...][128:192]) + su1b[...]


def _tcb_body(bonds_ref, bfc1w, bfc1b, bfc2w, bfc2b, bf_ref):
    x = bonds_ref[...]
    h = _relu(_mm(x, bfc1w[...]) + bfc1b[...])
    bfeat = _relu(_mm(h, bfc2w[...]) + bfc2b[...])
    bf_ref[...] = jnp.concatenate([bfeat[:BLKE // 2], bfeat[BLKE // 2:]],
                                  axis=1)


def _tcc_body(ab_ref, bfp_ref, g2b_ref, p4g_ref,
              bu1w, bu2w, bu2b, bu3w, bu3b,
              bout_ref, bnew_ref, bpool_ref, bcnt_ref):
    i = pl.program_id(0)
    bfp = bfp_ref[...]
    bfeat = jnp.concatenate([bfp[:, 0:64], bfp[:, 64:128]], axis=0)
    g2b = g2b_ref[0, 0, :]
    oh = (lax.broadcasted_iota(jnp.int32, (BLKE, N_GRAPHS), 1)
          == g2b[:, None]).astype(bf16)
    oht = (lax.broadcasted_iota(jnp.int32, (N_GRAPHS, BLKE), 0)
           == g2b[None, :])
    gterm = jnp.dot(oh, p4g_ref[...].astype(bf16), preferred_element_type=f32)
    w3 = bu1w[...][128:192]
    ab = ab_ref[...]
    h1 = _relu(ab[:, 0:64] + ab[:, 64:128] + _mm(bfeat, w3) + gterm)
    h2 = _relu(_mm(h1, bu2w[...]) + bu2b[...])
    bn = _relu(_mm(h2, bu3w[...]) + bu3b[...])
    bout_ref[...] = (bn + bfeat).T
    bnew_ref[...] = jnp.concatenate([bn[:BLKE // 2], bn[BLKE // 2:]], axis=1)

    @pl.when(i == 0)
    def _():
        bpool_ref[...] = jnp.zeros_like(bpool_ref)
        bcnt_ref[...] = jnp.zeros_like(bcnt_ref)

    bpool_ref[...] += jnp.dot(oht.astype(bf16), bn.astype(bf16),
                              preferred_element_type=f32)
    bcnt_ref[...] += jnp.sum(oht.astype(f32), axis=1, keepdims=True)


def _tce_body(nsum_ref, ncnt_ref, sfeat_ref, g2s_ref, psu_ref,
              su1w, su2w, su2b, su3w, su3b,
              bpool_ref, bcnt_ref, gfeat_ref,
              xu1w, xu1b, xu2w, xu2b, xu3w, xu3b,
              sout_ref, spool_ref, scnt_ref, gout_ref):
    i = pl.program_id(0)
    nsum = nsum_ref[0] + nsum_ref[1]
    cnt = ncnt_ref[0, :, 0:1] + ncnt_ref[1, :, 0:1]
    bp = nsum / jnp.maximum(cnt, 1.0)
    sf = sfeat_ref[...]
    g2s = g2s_ref[0, 0, :]
    oh = (lax.broadcasted_iota(jnp.int32, (BLKN, N_GRAPHS), 1)
          == g2s[:, None]).astype(bf16)
    oht = (lax.broadcasted_iota(jnp.int32, (N_GRAPHS, BLKN), 0)
           == g2s[None, :])
    w = su1w[...]
    gterm = jnp.dot(oh, psu_ref[...].astype(bf16), preferred_element_type=f32)
    h = _relu(_mm(bp, w[0:64]) + _mm(sf, w[64:128]) + gterm)
    h = _relu(_mm(h, su2w[...]) + su2b[...])
    sn = _relu(_mm(h, su3w[...]) + su3b[...])
    sout_ref[...] = sn + sf

    @pl.when(i == 0)
    def _():
        spool_ref[...] = jnp.zeros_like(spool_ref)
        scnt_ref[...] = jnp.zeros_like(scnt_ref)

    spool_ref[...] += jnp.dot(oht.astype(bf16), sn.astype(bf16),
                              preferred_element_type=f32)
    scnt_ref[...] += jnp.sum(oht.astype(f32), axis=1, keepdims=True)

    @pl.when(i == pl.num_programs(0) - 1)
    def _():
        bp2 = bpool_ref[...] / jnp.maximum(bcnt_ref[...], 1.0)
        sp2 = spool_ref[...] / jnp.maximum(scnt_ref[...], 1.0)
        gf = gfeat_ref[...]
        wx = xu1w[...]
        hx = _relu(_mm(bp2, wx[0:64]) + _mm(sp2, wx[64:128])
                   + _mm(gf, wx[128:192]) + xu1b[...])
        hx = _relu(_mm(hx, xu2w[...]) + xu2b[...])
        gn = _relu(_mm(hx, xu3w[...]) + xu3b[...])
        gout_ref[...] = gn + gf


def _sc_gather(table, idx):
    n_idx = idx.shape[1]
    n_rows = table.shape[0]
    srows = n_rows // 16
    mesh = plsc.VectorSubcoreMesh(core_axis_name="c", subcore_axis_name="s")

    def body(table_hbm, idx_hbm, out_hbm, table_sh):
        sid = lax.axis_index("s")
        pltpu.sync_copy(table_hbm.at[pl.ds(sid * srows, srows)],
                        table_sh.at[pl.ds(sid * srows, srows)])
        plsc.subcore_barrier()

        def inner(i_vmem, o_vmem):
            pltpu.sync_copy(table_sh.at[i_vmem.at[0]], o_vmem)

        pltpu.emit_pipeline(
            inner,
            grid=(n_idx // GW,),
            in_specs=[pl.BlockSpec((1, GW), lambda i: (0, i))],
            out_specs=[pl.BlockSpec((GW, 64), lambda i: (i, 0))],
            core_axis_name=("c", "s"),
            dimension_semantics=(pltpu.PARALLEL,),
        )(idx_hbm, out_hbm)

    fn = pl.kernel(body, out_type=jax.ShapeDtypeStruct((n_idx, 64), f32),
                   mesh=mesh, compiler_params=_SC_PARAMS,
                   scratch_types=[pltpu.VMEM_SHARED((n_rows, 64), f32)])
    return fn(table, idx)


def _sc_scatter(bnew, idx):
    mesh = plsc.VectorSubcoreMesh(core_axis_name="c", subcore_axis_name="s")
    NSUB = 16
    ROWS = N_NODES // NSUB

    def body(bnew_hbm, idx_hbm, nsum_hbm, ncnt_hbm,
             acc_sh, cnt_sh, ones_v, zrow_v, zrow16_v, sem1, sem2):
        cid = lax.axis_index("c")
        sid = lax.axis_index("s")

        @pl.loop(0, GW)
        def _(r):
            ones_v[pl.ds(r, 1), pl.ds(0, 16)] = jnp.ones((1, 16), f32)

        @pl.loop(0, 125)
        def _(r):
            @pl.loop(0, 64, step=16)
            def _(c2):
                zrow_v[pl.ds(r, 1), pl.ds(c2, 16)] = jnp.zeros((1, 16), f32)

            zrow16_v[pl.ds(r, 1), pl.ds(0, 16)] = jnp.zeros((1, 16), f32)

        @pl.loop(0, 5)
        def _(k):
            base = sid * ROWS + k * 125
            pltpu.sync_copy(zrow_v, acc_sh.at[pl.ds(base, 125)])
            pltpu.sync_copy(zrow16_v, cnt_sh.at[pl.ds(base, 125)])

        plsc.subcore_barrier()

        def inner(v_vmem, i_vmem):
            h1 = pltpu.async_copy(v_vmem, acc_sh.at[i_vmem.at[0]], sem1,
                                  add=True)
            h2 = pltpu.async_copy(ones_v, cnt_sh.at[i_vmem.at[0]], sem2,
                                  add=True)
            h1.wait()
            h2.wait()

        pltpu.emit_pipeline(
            inner,
            grid=(N_EDGES // GW,),
            in_specs=[pl.BlockSpec((GW, 64), lambda i: (i, 0)),
                      pl.BlockSpec((1, GW), lambda i: (0, i))],
            out_specs=[],
            core_axis_name=("c", "s"),
            dimension_semantics=(pltpu.PARALLEL,),
        )(bnew_hbm, idx_hbm)

        plsc.subcore_barrier()

        pltpu.sync_copy(acc_sh.at[pl.ds(sid * ROWS, ROWS)],
                        nsum_hbm.at[cid].at[pl.ds(sid * ROWS, ROWS)])
        pltpu.sync_copy(cnt_sh.at[pl.ds(sid * ROWS, ROWS)],
                        ncnt_hbm.at[cid].at[pl.ds(sid * ROWS, ROWS)])

    fn = pl.kernel(
        body,
        out_type=[jax.ShapeDtypeStruct((2, N_NODES, 64), f32),
                  jax.ShapeDtypeStruct((2, N_NODES, 16), f32)],
        mesh=mesh,
        compiler_params=_SC_PARAMS,
        scratch_types=[
            pltpu.VMEM_SHARED((N_NODES, 64), f32),
            pltpu.VMEM_SHARED((N_NODES, 16), f32),
            pltpu.VMEM((GW, 16), f32),
            pltpu.VMEM((125, 64), f32),
            pltpu.VMEM((125, 16), f32),
            pltpu.SemaphoreType.DMA,
            pltpu.SemaphoreType.DMA,
        ],
    )
    return fn(bnew, idx)


def kernel(sites, bonds, states, indices1, indices2, graph_to_sites,
           graph_to_bonds, bfc1_W, bfc1_b, bfc2_W, bfc2_b, sfc1_W, sfc1_b,
           sfc2_W, sfc2_b, gfc1_W, gfc1_b, gfc2_W, gfc2_b, bu1_W, bu1_b,
           bu2_W, bu2_b, bu3_W, bu3_b, su1_W, su1_b, su2_W, su2_b, su3_W,
           su3_b, xu1_W, xu1_b, xu2_W, xu2_b, xu3_W, xu3_b):
    r2 = lambda b: b.reshape(1, -1)
    i32 = jnp.int32
    idx1 = indices1.astype(i32)
    idx2 = indices2.astype(i32)
    g2b = graph_to_bonds.astype(i32)
    g2s = graph_to_sites.astype(i32)

    n_nblk = N_NODES // BLKN
    n_eblk = N_EDGES // BLKE
    const = lambda shp: pl.BlockSpec(shp, lambda i: tuple(0 for _ in shp))

    sfeat, T, gfeat, p4g, psu = pl.pallas_call(
        _tca_body,
        grid=(n_nblk,),
        in_specs=[
            pl.BlockSpec((BLKN, 128), lambda i: (i, 0)),
            const((N_GRAPHS, 128)),
            const((128, 64)), const((1, 64)), const((64, 64)), const((1, 64)),
            const((128, 64)), const((1, 64)), const((64, 64)), const((1, 64)),
            const((256, 64)), const((1, 64)), const((192, 64)), const((1, 64)),
        ],
        out_specs=[
            pl.BlockSpec((BLKN, 64), lambda i: (i, 0)),
            pl.BlockSpec((BLKN, 128), lambda i: (i, 0)),
            const((N_GRAPHS, 64)),
            const((N_GRAPHS, 64)),
            const((N_GRAPHS, 64)),
        ],
        out_shape=[
            jax.ShapeDtypeStruct((N_NODES, 64), f32),
            jax.ShapeDtypeStruct((N_NODES, 128), f32),
            jax.ShapeDtypeStruct((N_GRAPHS, 64), f32),
            jax.ShapeDtypeStruct((N_GRAPHS, 64), f32),
            jax.ShapeDtypeStruct((N_GRAPHS, 64), f32),
        ],
    )(sites, states, sfc1_W, r2(sfc1_b), sfc2_W, r2(sfc2_b),
      gfc1_W, r2(gfc1_b), gfc2_W, r2(gfc2_b), bu1_W, r2(bu1_b),
      su1_W, r2(su1_b))

    table = T.reshape(2 * N_NODES, 64)
    idx_all = jnp.stack([2 * idx1, 2 * idx2 + 1], axis=1).reshape(1, -1)
    ab = _sc_gather(table, idx_all).reshape(N_EDGES, 128)

    bfp = pl.pallas_call(
        _tcb_body,
        grid=(n_eblk,),
        in_specs=[
            pl.BlockSpec((BLKE, 128), lambda i: (i, 0)),
            const((128, 64)), const((1, 64)), const((64, 64)), const((1, 64)),
        ],
        out_specs=pl.BlockSpec((BLKE // 2, 128), lambda i: (i, 0)),
        out_shape=jax.ShapeDtypeStruct((N_EDGES // 2, 128), f32),
    )(bonds, bfc1_W, r2(bfc1_b), bfc2_W, r2(bfc2_b))

    g2b3 = g2b.reshape(n_eblk, 1, BLKE)
    bout, bnew, bpool_s, bcnt = pl.pallas_call(
        _tcc_body,
        grid=(n_eblk,),
        in_specs=[
            pl.BlockSpec((BLKE, 128), lambda i: (i, 0)),
            pl.BlockSpec((BLKE // 2, 128), lambda i: (i, 0)),
            pl.BlockSpec((1, 1, BLKE), lambda i: (i, 0, 0)),
            const((N_GRAPHS, 64)),
            const((256, 64)), const((64, 64)), const((1, 64)),
            const((64, 64)), const((1, 64)),
        ],
        out_specs=[
            pl.BlockSpec((64, BLKE), lambda i: (0, i)),
            pl.BlockSpec((BLKE // 2, 128), lambda i: (i, 0)),
            const((N_GRAPHS, 64)),
            const((N_GRAPHS, 1)),
        ],
        out_shape=[
            jax.ShapeDtypeStruct((64, N_EDGES), f32),
            jax.ShapeDtypeStruct((N_EDGES // 2, 128), f32),
            jax.ShapeDtypeStruct((N_GRAPHS, 64), f32),
            jax.ShapeDtypeStruct((N_GRAPHS, 1), f32),
        ],
    )(ab, bfp, g2b3, p4g, bu1_W, bu2_W, r2(bu2_b), bu3_W, r2(bu3_b))
    bout = bout.T

    idx_sc = (idx1.reshape(n_eblk, 2, BLKE // 2)
              .transpose(0, 2, 1).reshape(1, -1))
    nsum, ncnt = _sc_scatter(bnew.reshape(N_EDGES, 64), idx_sc)

    g2s3 = g2s.reshape(n_nblk, 1, BLKN)
    sout, spool_s, scnt, gout = pl.pallas_call(
        _tce_body,
        grid=(n_nblk,),
        in_specs=[
            pl.BlockSpec((2, BLKN, 64), lambda i: (0, i, 0)),
            pl.BlockSpec((2, BLKN, 16), lambda i: (0, i, 0)),
            pl.BlockSpec((BLKN, 64), lambda i: (i, 0)),
            pl.BlockSpec((1, 1, BLKN), lambda i: (i, 0, 0)),
            const((N_GRAPHS, 64)),
            const((192, 64)), const((64, 64)), const((1, 64)),
            const((64, 64)), const((1, 64)),
            const((N_GRAPHS, 64)), const((N_GRAPHS, 1)),
            const((N_GRAPHS, 64)),
            const((192, 64)), const((1, 64)), const((64, 64)), const((1, 64)),
            const((64, 64)), const((1, 64)),
        ],
        out_specs=[
            pl.BlockSpec((BLKN, 64), lambda i: (i, 0)),
            const((N_GRAPHS, 64)),
            const((N_GRAPHS, 1)),
            const((N_GRAPHS, 64)),
        ],
        out_shape=[
            jax.ShapeDtypeStruct((N_NODES, 64), f32),
            jax.ShapeDtypeStruct((N_GRAPHS, 64), f32),
            jax.ShapeDtypeStruct((N_GRAPHS, 1), f32),
            jax.ShapeDtypeStruct((N_GRAPHS, 64), f32),
        ],
    )(nsum, ncnt, sfeat, g2s3, psu, su1_W, su2_W, r2(su2_b),
      su3_W, r2(su3_b), bpool_s, bcnt, gfeat, xu1_W, r2(xu1_b),
      xu2_W, r2(xu2_b), xu3_W, r2(xu3_b))

    return sout, bout, gout
